# baseline (speedup 1.0000x reference)
import functools
import jax
import jax.numpy as jnp
import numpy as np
from jax import lax
from jax.experimental import pallas as pl
from jax.experimental.pallas import tpu as pltpu
from jax.experimental.pallas import tpu_sc as plsc

N = 10000
E = 320000
D = 128
HEADS = 8
OPH = 16
LAYERS = 2
HOPS = 2
DECAY = [float(np.exp(-0.5 * k)) for k in range(HOPS)]

NC = 2
NS = 16
NW = NC * NS
LN = 16

NPAD = 10240
DUMMY = N
ROWS_PT = NPAD // NS

ECHUNK = 128
ECONV = 64
CSTEPS = 162
CPHASES = 3
PHSTEPS = CSTEPS // CPHASES
UNROLL = 8
EPW = ECONV * CSTEPS
EPAD = EPW * NW

K1ROWS = 79
K1PAD = NW * K1ROWS * ECHUNK

_mesh = plsc.VectorSubcoreMesh(core_axis_name="c", subcore_axis_name="s")
_sc_params = pltpu.CompilerParams(use_tc_tiling_on_sc=False,
                                  needs_layout_passes=False)


def _wid():
    return lax.axis_index("s") * NC + lax.axis_index("c")


def _k1_body(src_hbm, dst_hbm, loc_out, nbrv, srcb, dstb, tmpb):
    w = _wid()
    def zz(i, _):
        nbrv[pl.ds(i * LN, LN)] = jnp.zeros((LN,), jnp.int32)
        return _
    lax.fori_loop(0, NPAD // LN, zz, 0)
    pltpu.sync_copy(src_hbm.at[pl.ds(w * K1ROWS, K1ROWS)], srcb)
    pltpu.sync_copy(dst_hbm.at[pl.ds(w * K1ROWS, K1ROWS)], dstb)

    iot = lax.iota(jnp.int32, LN)
    idxm1 = jnp.maximum(iot - 1, 0)

    def row(r, _):
        for g in range(ECHUNK // LN):
            sv = srcb[r, pl.ds(g * LN, LN)]
            dv = dstb[r, pl.ds(g * LN, LN)]
            key = sv * 16384 + dv
            sk, dsort = plsc.sort_key_val(key, dv, descending=True)
            ssort = lax.shift_right_logical(sk, 14)
            tmpb[...] = ssort
            prev = plsc.load_gather(tmpb, [idxm1])
            first = (ssort != prev) | (iot == 0)
            cur = plsc.load_gather(nbrv, [ssort])
            plsc.store_scatter(nbrv, [ssort], jnp.maximum(cur, dsort),
                               mask=first)
        return _
    lax.fori_loop(0, K1ROWS, row, 0)
    pltpu.sync_copy(nbrv, loc_out.at[w])


def _k2_body(loc_hbm, nbr_out, buf, mrg):
    w = _wid()
    cols = NPAD // NW
    pltpu.sync_copy(loc_hbm.at[:, pl.ds(w * cols, cols)], buf)

    def chunk(cix, _):
        def red(r, m):
            return jnp.maximum(m, buf[r, pl.ds(cix * LN, LN)])
        m = lax.fori_loop(1, NW, red, buf[0, pl.ds(cix * LN, LN)])
        mrg[pl.ds(cix * LN, LN)] = m
        return _
    lax.fori_loop(0, cols // LN, chunk, 0)
    pltpu.sync_copy(mrg, nbr_out.at[pl.ds(w * cols, cols)])


def _k3_body(nbr_hbm, dst_hbm, dst2_out, nbrv, dstb, d2b):
    w = _wid()
    pltpu.sync_copy(nbr_hbm, nbrv)
    pltpu.sync_copy(dst_hbm.at[pl.ds(w * K1ROWS, K1ROWS)], dstb)

    def row(r, _):
        for g in range(ECHUNK // LN):
            dv = dstb[r, pl.ds(g * LN, LN)]
            d2b[r, pl.ds(g * LN, LN)] = plsc.load_gather(nbrv, [dv])
        return _
    lax.fori_loop(0, K1ROWS, row, 0)
    pltpu.sync_copy(d2b, dst2_out.at[pl.ds(w * K1ROWS, K1ROWS)])


_k1 = pl.kernel(
    _k1_body,
    out_type=jax.ShapeDtypeStruct((NW, NPAD), jnp.int32),
    mesh=_mesh,
    compiler_params=_sc_params,
    scratch_types=[
        pltpu.VMEM((NPAD,), jnp.int32),
        pltpu.VMEM((K1ROWS, ECHUNK), jnp.int32),
        pltpu.VMEM((K1ROWS, ECHUNK), jnp.int32),
        pltpu.VMEM((LN,), jnp.int32),
    ],
)

_k2 = pl.kernel(
    _k2_body,
    out_type=jax.ShapeDtypeStruct((NPAD,), jnp.int32),
    mesh=_mesh,
    compiler_params=_sc_params,
    scratch_types=[
        pltpu.VMEM((NW, NPAD // NW), jnp.int32),
        pltpu.VMEM((NPAD // NW,), jnp.int32),
    ],
)

_k3 = pl.kernel(
    _k3_body,
    out_type=jax.ShapeDtypeStruct((NW * K1ROWS, ECHUNK), jnp.int32),
    mesh=_mesh,
    compiler_params=_sc_params,
    scratch_types=[
        pltpu.VMEM((NPAD,), jnp.int32),
        pltpu.VMEM((K1ROWS, ECHUNK), jnp.int32),
        pltpu.VMEM((K1ROWS, ECHUNK), jnp.int32),
    ],
)


def _conv_body(src_hbm, dst_hbm, ssrc_hbm, sdst_hbm, h2_hbm,
               num_out, den_out,
               srcb, dstb, sgs0, sgs1, sgd0, sgd1, h2g0, h2g1, wnum, wden,
               numsh, densh, sem0, sem1):
    c = lax.axis_index("c")
    s = lax.axis_index("s")
    w = s * NC + c
    sgs = (sgs0, sgs1)
    sgd = (sgd0, sgd1)
    h2g = (h2g0, h2g1)
    sem = (sem0, sem1)

    def zrow(i, _):
        for g in range(D // LN):
            wnum[i, pl.ds(g * LN, LN)] = jnp.zeros((LN,), jnp.float32)
        wden[i, pl.ds(0, LN)] = jnp.zeros((LN,), jnp.float32)
        return _
    lax.fori_loop(0, ECONV, zrow, 0)
    for b in range(ROWS_PT // ECONV):
        r = s * ROWS_PT + b * ECONV
        pltpu.sync_copy(wnum, numsh.at[pl.ds(r, ECONV)])
        pltpu.sync_copy(wden, densh.at[pl.ds(r, ECONV)])
    plsc.subcore_barrier()

    def issue(j, b):
        c1 = pltpu.async_copy(ssrc_hbm.at[srcb.at[j]], sgs[b], sem[b])
        c2 = pltpu.async_copy(sdst_hbm.at[dstb.at[j]], sgd[b], sem[b])
        c3 = pltpu.async_copy(h2_hbm.at[srcb.at[j]], h2g[b], sem[b])
        return (c1, c2, c3)

    def process(j, b, cps):
        for cp in cps:
            cp.wait()
        bs, bd, bh = sgs[b], sgd[b], h2g[b]

        @plsc.parallel_loop(0, ECONV, unroll=UNROLL)
        def _edges(i):
            ev = bs[i] + bd[i]
            ev = jnp.where(ev >= 0, ev, 0.2 * ev)
            wv = jnp.exp(ev)
            wden[i] = wv
            for h in range(HEADS):
                wh = wv[h]
                wnum[i, pl.ds(h * OPH, OPH)] = (
                    bh[i, pl.ds(h * OPH, OPH)] * wh)

        idxd = dstb.at[j]
        pltpu.sync_copy(wnum, numsh.at[idxd], add=True)
        pltpu.sync_copy(wden, densh.at[idxd], add=True)

    for p in range(CPHASES):
        base = w * CSTEPS + p * PHSTEPS
        pltpu.sync_copy(src_hbm.at[pl.ds(base, PHSTEPS)], srcb)
        pltpu.sync_copy(dst_hbm.at[pl.ds(base, PHSTEPS)], dstb)

        cps0 = issue(0, 0)

        def pair(t, _):
            j0 = t * 2
            cb = issue(j0 + 1, 1)
            process(j0, 0, cps0)

            @pl.when(t < PHSTEPS // 2 - 1)
            def _issue_next():
                issue(j0 + 2, 0)

            process(j0 + 1, 1, cb)
            return _
        lax.fori_loop(0, PHSTEPS // 2, pair, 0)

    plsc.subcore_barrier()
    pltpu.sync_copy(numsh.at[pl.ds(s * ROWS_PT, ROWS_PT)],
                    num_out.at[c, pl.ds(s * ROWS_PT, ROWS_PT)])
    pltpu.sync_copy(densh.at[pl.ds(s * ROWS_PT, ROWS_PT)],
                    den_out.at[c, pl.ds(s * ROWS_PT, ROWS_PT)])


_conv = pl.kernel(
    _conv_body,
    out_type=(
        jax.ShapeDtypeStruct((NC, NPAD, D), jnp.float32),
        jax.ShapeDtypeStruct((NC, NPAD, LN), jnp.float32),
    ),
    mesh=_mesh,
    compiler_params=_sc_params,
    scratch_types=[
        pltpu.VMEM((PHSTEPS, ECONV), jnp.int32),
        pltpu.VMEM((PHSTEPS, ECONV), jnp.int32),
        pltpu.VMEM((ECONV, LN), jnp.float32),
        pltpu.VMEM((ECONV, LN), jnp.float32),
        pltpu.VMEM((ECONV, LN), jnp.float32),
        pltpu.VMEM((ECONV, LN), jnp.float32),
        pltpu.VMEM((ECONV, D), jnp.float32),
        pltpu.VMEM((ECONV, D), jnp.float32),
        pltpu.VMEM((ECONV, D), jnp.float32),
        pltpu.VMEM((ECONV, LN), jnp.float32),
        pltpu.VMEM_SHARED((NPAD, D), jnp.float32),
        pltpu.VMEM_SHARED((NPAD, LN), jnp.float32),
        pltpu.SemaphoreType.DMA,
        pltpu.SemaphoreType.DMA,
    ],
)


_BLK = 256
_GRID = NPAD // _BLK


def _tc0_body(x_ref, w_ref, b_ref, o_ref):
    v = jnp.dot(x_ref[...], w_ref[...], preferred_element_type=jnp.float32)
    v = v + b_ref[...]
    o_ref[...] = jnp.where(v >= 0, v, 0.01 * v)


def _tc0(x, W1, b1):
    return pl.pallas_call(
        _tc0_body,
        grid=(_GRID,),
        in_specs=[
            pl.BlockSpec((_BLK, D), lambda i: (i, 0)),
            pl.BlockSpec((D, D), lambda i: (0, 0)),
            pl.BlockSpec((1, D), lambda i: (0, 0)),
        ],
        out_specs=pl.BlockSpec((_BLK, D), lambda i: (i, 0)),
        out_shape=jax.ShapeDtypeStruct((NPAD, D), jnp.float32),
    )(x, W1, b1)


def _tca_body(h_ref, wg0, as0, ad0, wg1, as1, ad1,
              h20, ss0, sd0, h21, ss1, sd1):
    h = h_ref[...]
    for wg, a_s, a_d, oh, oss, osd in ((wg0, as0, ad0, h20, ss0, sd0),
                                       (wg1, as1, ad1, h21, ss1, sd1)):
        h2 = jnp.dot(h, wg[...], preferred_element_type=jnp.float32)
        oh[...] = h2
        ts = (h2 * a_s[...]).reshape(_BLK, HEADS, OPH).sum(-1)
        td = (h2 * a_d[...]).reshape(_BLK, HEADS, OPH).sum(-1)
        oss[...] = jnp.concatenate([ts, jnp.zeros_like(ts)], axis=1)
        osd[...] = jnp.concatenate([td, jnp.zeros_like(td)], axis=1)


def _tca(h, wg0, as0, ad0, wg1, as1, ad1):
    row = pl.BlockSpec((_BLK, D), lambda i: (i, 0))
    srow = pl.BlockSpec((_BLK, LN), lambda i: (i, 0))
    wfull = pl.BlockSpec((D, D), lambda i: (0, 0))
    vfull = pl.BlockSpec((1, D), lambda i: (0, 0))
    return pl.pallas_call(
        _tca_body,
        grid=(_GRID,),
        in_specs=[row, wfull, vfull, vfull, wfull, vfull, vfull],
        out_specs=[row, srow, srow, row, srow, srow],
        out_shape=[
            jax.ShapeDtypeStruct((NPAD, D), jnp.float32),
            jax.ShapeDtypeStruct((NPAD, LN), jnp.float32),
            jax.ShapeDtypeStruct((NPAD, LN), jnp.float32),
            jax.ShapeDtypeStruct((NPAD, D), jnp.float32),
            jax.ShapeDtypeStruct((NPAD, LN), jnp.float32),
            jax.ShapeDtypeStruct((NPAD, LN), jnp.float32),
        ],
    )(h, wg0, as0, ad0, wg1, as1, ad1)


def _tcb_body(n0, d0, n1, d1, gb0, dw0, db0, gb1, dw1, db1,
              lng, lnb, hres, hout):
    acc = jnp.zeros((_BLK, D), jnp.float32)
    for k, (na, da, gb, dw, db) in enumerate(((n0, d0, gb0, dw0, db0),
                                              (n1, d1, gb1, dw1, db1))):
        n = na[0] + na[1]
        d8 = da[0, :, :HEADS] + da[1, :, :HEADS]
        g = n.reshape(_BLK, HEADS, OPH) / (d8[..., None] + 1e-16)
        g = g.reshape(_BLK, D) + gb[...]
        xk = jnp.dot(g, dw[...], preferred_element_type=jnp.float32) + db[...]
        xk = jnp.where(xk >= 0, xk, 0.01 * xk)
        acc = acc + DECAY[k] * xk
    mu = jnp.mean(acc, axis=-1, keepdims=True)
    var = jnp.mean((acc - mu) ** 2, axis=-1, keepdims=True)
    xl = (acc - mu) * lax.rsqrt(var + 1e-5) * lng[...] + lnb[...]
    hout[...] = xl + hres[...]


def _tcb(num0, den0, num1, den1, gb0, dw0, db0, gb1, dw1, db1, lng, lnb, h):
    nspec = pl.BlockSpec((NC, _BLK, D), lambda i: (0, i, 0))
    dspec = pl.BlockSpec((NC, _BLK, LN), lambda i: (0, i, 0))
    row = pl.BlockSpec((_BLK, D), lambda i: (i, 0))
    wfull = pl.BlockSpec((D, D), lambda i: (0, 0))
    vfull = pl.BlockSpec((1, D), lambda i: (0, 0))
    return pl.pallas_call(
        _tcb_body,
        grid=(_GRID,),
        in_specs=[nspec, dspec, nspec, dspec,
                  vfull, wfull, vfull, vfull, wfull, vfull,
                  vfull, vfull, row],
        out_specs=row,
        out_shape=jax.ShapeDtypeStruct((NPAD, D), jnp.float32),
    )(num0, den0, num1, den1, gb0, dw0, db0, gb1, dw1, db1, lng, lnb, h)


def _pad_edges(a, total, fill, width):
    return jnp.concatenate(
        [a, jnp.full((total - a.shape[0],), fill, jnp.int32)]
    ).reshape(-1, width)


def kernel(x, edge_index, edge_type, genre, genre_mask, W1, b1, gat_W,
           att_src, att_dst, gat_b, dec_W, dec_b, ln_g, ln_b):
    del edge_type, genre, genre_mask
    src = edge_index[0].astype(jnp.int32)
    dst = edge_index[1].astype(jnp.int32)
    loop = jnp.arange(N, dtype=jnp.int32)

    srcE = _pad_edges(src, K1PAD, DUMMY, ECHUNK)
    dstE = _pad_edges(dst, K1PAD, 0, ECHUNK)
    loc = _k1(srcE, dstE)
    nbr = _k2(loc)
    dst2 = _k3(nbr, dstE).reshape(-1)[:E]

    src_c = _pad_edges(jnp.concatenate([src, loop]), EPAD, DUMMY, ECONV)
    dst_c0 = _pad_edges(jnp.concatenate([dst, loop]), EPAD, DUMMY, ECONV)
    dst_c1 = _pad_edges(jnp.concatenate([dst2, loop]), EPAD, DUMMY, ECONV)

    xp = jnp.zeros((NPAD, D), jnp.float32).at[:N].set(x)
    h = _tc0(xp, W1, b1.reshape(1, D))

    r2 = lambda a: a.reshape(1, D)
    for l in range(LAYERS):
        h20, ss0, sd0, h21, ss1, sd1 = _tca(
            h, gat_W[l, 0], r2(att_src[l, 0]), r2(att_dst[l, 0]),
            gat_W[l, 1], r2(att_src[l, 1]), r2(att_dst[l, 1]))
        num0, den0 = _conv(src_c, dst_c0, ss0, sd0, h20)
        num1, den1 = _conv(src_c, dst_c1, ss1, sd1, h21)
        h = _tcb(num0, den0, num1, den1,
                 r2(gat_b[l, 0]), dec_W[l, 0], r2(dec_b[l, 0]),
                 r2(gat_b[l, 1]), dec_W[l, 1], r2(dec_b[l, 1]),
                 r2(ln_g[l]), r2(ln_b[l]), h)
    return h[:N]


# speedup vs baseline: 1.0265x; 1.0265x over previous
import functools
import jax
import jax.numpy as jnp
import numpy as np
from jax import lax
from jax.experimental import pallas as pl
from jax.experimental.pallas import tpu as pltpu
from jax.experimental.pallas import tpu_sc as plsc

N = 10000
E = 320000
D = 128
HEADS = 8
OPH = 16
LAYERS = 2
HOPS = 2
DECAY = [float(np.exp(-0.5 * k)) for k in range(HOPS)]

NC = 2
NS = 16
NW = NC * NS
LN = 16

NPAD = 10240
DUMMY = N
ROWS_PT = NPAD // NS

ECHUNK = 128
ECONV = 64
CSTEPS = 162
CPHASES = 3
PHSTEPS = CSTEPS // CPHASES
UNROLL = 4
EPW = ECONV * CSTEPS
EPAD = EPW * NW

K1ROWS = 79
K1PAD = NW * K1ROWS * ECHUNK

_mesh = plsc.VectorSubcoreMesh(core_axis_name="c", subcore_axis_name="s")
_sc_params = pltpu.CompilerParams(use_tc_tiling_on_sc=False,
                                  needs_layout_passes=False)


def _wid():
    return lax.axis_index("s") * NC + lax.axis_index("c")


def _k1_body(src_hbm, dst_hbm, loc_out, nbrv, srcb, dstb, tmpb):
    w = _wid()
    def zz(i, _):
        nbrv[pl.ds(i * LN, LN)] = jnp.zeros((LN,), jnp.int32)
        return _
    lax.fori_loop(0, NPAD // LN, zz, 0)
    pltpu.sync_copy(src_hbm.at[pl.ds(w * K1ROWS, K1ROWS)], srcb)
    pltpu.sync_copy(dst_hbm.at[pl.ds(w * K1ROWS, K1ROWS)], dstb)

    iot = lax.iota(jnp.int32, LN)
    idxm1 = jnp.maximum(iot - 1, 0)

    def row(r, _):
        for g in range(ECHUNK // LN):
            sv = srcb[r, pl.ds(g * LN, LN)]
            dv = dstb[r, pl.ds(g * LN, LN)]
            key = sv * 16384 + dv
            sk, dsort = plsc.sort_key_val(key, dv, descending=True)
            ssort = lax.shift_right_logical(sk, 14)
            tmpb[...] = ssort
            prev = plsc.load_gather(tmpb, [idxm1])
            first = (ssort != prev) | (iot == 0)
            cur = plsc.load_gather(nbrv, [ssort])
            plsc.store_scatter(nbrv, [ssort], jnp.maximum(cur, dsort),
                               mask=first)
        return _
    lax.fori_loop(0, K1ROWS, row, 0)
    pltpu.sync_copy(nbrv, loc_out.at[w])


def _k2_body(loc_hbm, nbr_out, buf, mrg):
    w = _wid()
    cols = NPAD // NW
    pltpu.sync_copy(loc_hbm.at[:, pl.ds(w * cols, cols)], buf)

    def chunk(cix, _):
        def red(r, m):
            return jnp.maximum(m, buf[r, pl.ds(cix * LN, LN)])
        m = lax.fori_loop(1, NW, red, buf[0, pl.ds(cix * LN, LN)])
        mrg[pl.ds(cix * LN, LN)] = m
        return _
    lax.fori_loop(0, cols // LN, chunk, 0)
    pltpu.sync_copy(mrg, nbr_out.at[pl.ds(w * cols, cols)])


def _k3_body(nbr_hbm, dst_hbm, dst2_out, nbrv, dstb, d2b):
    w = _wid()
    pltpu.sync_copy(nbr_hbm, nbrv)
    pltpu.sync_copy(dst_hbm.at[pl.ds(w * K1ROWS, K1ROWS)], dstb)

    def row(r, _):
        for g in range(ECHUNK // LN):
            dv = dstb[r, pl.ds(g * LN, LN)]
            d2b[r, pl.ds(g * LN, LN)] = plsc.load_gather(nbrv, [dv])
        return _
    lax.fori_loop(0, K1ROWS, row, 0)
    pltpu.sync_copy(d2b, dst2_out.at[pl.ds(w * K1ROWS, K1ROWS)])


_k1 = pl.kernel(
    _k1_body,
    out_type=jax.ShapeDtypeStruct((NW, NPAD), jnp.int32),
    mesh=_mesh,
    compiler_params=_sc_params,
    scratch_types=[
        pltpu.VMEM((NPAD,), jnp.int32),
        pltpu.VMEM((K1ROWS, ECHUNK), jnp.int32),
        pltpu.VMEM((K1ROWS, ECHUNK), jnp.int32),
        pltpu.VMEM((LN,), jnp.int32),
    ],
)

_k2 = pl.kernel(
    _k2_body,
    out_type=jax.ShapeDtypeStruct((NPAD,), jnp.int32),
    mesh=_mesh,
    compiler_params=_sc_params,
    scratch_types=[
        pltpu.VMEM((NW, NPAD // NW), jnp.int32),
        pltpu.VMEM((NPAD // NW,), jnp.int32),
    ],
)

_k3 = pl.kernel(
    _k3_body,
    out_type=jax.ShapeDtypeStruct((NW * K1ROWS, ECHUNK), jnp.int32),
    mesh=_mesh,
    compiler_params=_sc_params,
    scratch_types=[
        pltpu.VMEM((NPAD,), jnp.int32),
        pltpu.VMEM((K1ROWS, ECHUNK), jnp.int32),
        pltpu.VMEM((K1ROWS, ECHUNK), jnp.int32),
    ],
)


def _conv_body(src_hbm, dst0_hbm, ss0_hbm, sd0_hbm, h20_hbm,
               dst1_hbm, ss1_hbm, sd1_hbm, h21_hbm,
               num0_out, den0_out, num1_out, den1_out,
               srcb, dstb, sgs0, sgs1, sgd0, sgd1, h2g0, h2g1, wnum, wden,
               numsh, densh, sem0, sem1):
    c = lax.axis_index("c")
    s = lax.axis_index("s")
    w = s * NC + c
    sgs = (sgs0, sgs1)
    sgd = (sgd0, sgd1)
    h2g = (h2g0, h2g1)
    sem = (sem0, sem1)

    def one_hop(dst_hbm, ssrc_hbm, sdst_hbm, h2_hbm, num_out, den_out):
        def zrow(i, _):
            for g in range(D // LN):
                wnum[i, pl.ds(g * LN, LN)] = jnp.zeros((LN,), jnp.float32)
            wden[i, pl.ds(0, LN)] = jnp.zeros((LN,), jnp.float32)
            return _
        lax.fori_loop(0, ECONV, zrow, 0)
        for b in range(ROWS_PT // ECONV):
            r = s * ROWS_PT + b * ECONV
            pltpu.sync_copy(wnum, numsh.at[pl.ds(r, ECONV)])
            pltpu.sync_copy(wden, densh.at[pl.ds(r, ECONV)])
        plsc.subcore_barrier()

        def issue(j, b):
            c1 = pltpu.async_copy(ssrc_hbm.at[srcb.at[j]], sgs[b], sem[b])
            c2 = pltpu.async_copy(sdst_hbm.at[dstb.at[j]], sgd[b], sem[b])
            c3 = pltpu.async_copy(h2_hbm.at[srcb.at[j]], h2g[b], sem[b])
            return (c1, c2, c3)

        def process(j, b, cps):
            for cp in cps:
                cp.wait()
            bs, bd, bh = sgs[b], sgd[b], h2g[b]

            @plsc.parallel_loop(0, ECONV, unroll=UNROLL)
            def _edges(i):
                ev = bs[i] + bd[i]
                ev = jnp.where(ev >= 0, ev, 0.2 * ev)
                wv = jnp.exp(ev)
                wden[i] = wv
                for h in range(HEADS):
                    wh = wv[h]
                    wnum[i, pl.ds(h * OPH, OPH)] = (
                        bh[i, pl.ds(h * OPH, OPH)] * wh)

            idxd = dstb.at[j]
            pltpu.sync_copy(wnum, numsh.at[idxd], add=True)
            pltpu.sync_copy(wden, densh.at[idxd], add=True)

        def phase(p, _):
            base = w * CSTEPS + p * PHSTEPS
            pltpu.sync_copy(src_hbm.at[pl.ds(base, PHSTEPS)], srcb)
            pltpu.sync_copy(dst_hbm.at[pl.ds(base, PHSTEPS)], dstb)

            cps0 = issue(0, 0)

            def pair(t, _):
                j0 = t * 2
                cb = issue(j0 + 1, 1)
                process(j0, 0, cps0)

                @pl.when(t < PHSTEPS // 2 - 1)
                def _issue_next():
                    issue(j0 + 2, 0)

                process(j0 + 1, 1, cb)
                return _
            lax.fori_loop(0, PHSTEPS // 2, pair, 0)
            return _
        lax.fori_loop(0, CPHASES, phase, 0)

        plsc.subcore_barrier()
        pltpu.sync_copy(numsh.at[pl.ds(s * ROWS_PT, ROWS_PT)],
                        num_out.at[c, pl.ds(s * ROWS_PT, ROWS_PT)])
        pltpu.sync_copy(densh.at[pl.ds(s * ROWS_PT, ROWS_PT)],
                        den_out.at[c, pl.ds(s * ROWS_PT, ROWS_PT)])
        plsc.subcore_barrier()

    one_hop(dst0_hbm, ss0_hbm, sd0_hbm, h20_hbm, num0_out, den0_out)
    one_hop(dst1_hbm, ss1_hbm, sd1_hbm, h21_hbm, num1_out, den1_out)


_conv = pl.kernel(
    _conv_body,
    out_type=(
        jax.ShapeDtypeStruct((NC, NPAD, D), jnp.float32),
        jax.ShapeDtypeStruct((NC, NPAD, LN), jnp.float32),
        jax.ShapeDtypeStruct((NC, NPAD, D), jnp.float32),
        jax.ShapeDtypeStruct((NC, NPAD, LN), jnp.float32),
    ),
    mesh=_mesh,
    compiler_params=_sc_params,
    scratch_types=[
        pltpu.VMEM((PHSTEPS, ECONV), jnp.int32),
        pltpu.VMEM((PHSTEPS, ECONV), jnp.int32),
        pltpu.VMEM((ECONV, LN), jnp.float32),
        pltpu.VMEM((ECONV, LN), jnp.float32),
        pltpu.VMEM((ECONV, LN), jnp.float32),
        pltpu.VMEM((ECONV, LN), jnp.float32),
        pltpu.VMEM((ECONV, D), jnp.float32),
        pltpu.VMEM((ECONV, D), jnp.float32),
        pltpu.VMEM((ECONV, D), jnp.float32),
        pltpu.VMEM((ECONV, LN), jnp.float32),
        pltpu.VMEM_SHARED((NPAD, D), jnp.float32),
        pltpu.VMEM_SHARED((NPAD, LN), jnp.float32),
        pltpu.SemaphoreType.DMA,
        pltpu.SemaphoreType.DMA,
    ],
)


_BLK = 256
_GRID = NPAD // _BLK


def _tc0_body(x_ref, w_ref, b_ref, o_ref):
    v = jnp.dot(x_ref[...], w_ref[...], preferred_element_type=jnp.float32)
    v = v + b_ref[...]
    o_ref[...] = jnp.where(v >= 0, v, 0.01 * v)


def _tc0(x, W1, b1):
    return pl.pallas_call(
        _tc0_body,
        grid=(_GRID,),
        in_specs=[
            pl.BlockSpec((_BLK, D), lambda i: (i, 0)),
            pl.BlockSpec((D, D), lambda i: (0, 0)),
            pl.BlockSpec((1, D), lambda i: (0, 0)),
        ],
        out_specs=pl.BlockSpec((_BLK, D), lambda i: (i, 0)),
        out_shape=jax.ShapeDtypeStruct((NPAD, D), jnp.float32),
    )(x, W1, b1)


def _tca_body(h_ref, wg0, as0, ad0, wg1, as1, ad1,
              h20, ss0, sd0, h21, ss1, sd1):
    h = h_ref[...]
    for wg, a_s, a_d, oh, oss, osd in ((wg0, as0, ad0, h20, ss0, sd0),
                                       (wg1, as1, ad1, h21, ss1, sd1)):
        h2 = jnp.dot(h, wg[...], preferred_element_type=jnp.float32)
        oh[...] = h2
        ts = (h2 * a_s[...]).reshape(_BLK, HEADS, OPH).sum(-1)
        td = (h2 * a_d[...]).reshape(_BLK, HEADS, OPH).sum(-1)
        oss[...] = jnp.concatenate([ts, jnp.zeros_like(ts)], axis=1)
        osd[...] = jnp.concatenate([td, jnp.zeros_like(td)], axis=1)


def _tca(h, wg0, as0, ad0, wg1, as1, ad1):
    row = pl.BlockSpec((_BLK, D), lambda i: (i, 0))
    srow = pl.BlockSpec((_BLK, LN), lambda i: (i, 0))
    wfull = pl.BlockSpec((D, D), lambda i: (0, 0))
    vfull = pl.BlockSpec((1, D), lambda i: (0, 0))
    return pl.pallas_call(
        _tca_body,
        grid=(_GRID,),
        in_specs=[row, wfull, vfull, vfull, wfull, vfull, vfull],
        out_specs=[row, srow, srow, row, srow, srow],
        out_shape=[
            jax.ShapeDtypeStruct((NPAD, D), jnp.float32),
            jax.ShapeDtypeStruct((NPAD, LN), jnp.float32),
            jax.ShapeDtypeStruct((NPAD, LN), jnp.float32),
            jax.ShapeDtypeStruct((NPAD, D), jnp.float32),
            jax.ShapeDtypeStruct((NPAD, LN), jnp.float32),
            jax.ShapeDtypeStruct((NPAD, LN), jnp.float32),
        ],
    )(h, wg0, as0, ad0, wg1, as1, ad1)


def _tcb_body(n0, d0, n1, d1, gb0, dw0, db0, gb1, dw1, db1,
              lng, lnb, hres, hout):
    acc = jnp.zeros((_BLK, D), jnp.float32)
    for k, (na, da, gb, dw, db) in enumerate(((n0, d0, gb0, dw0, db0),
                                              (n1, d1, gb1, dw1, db1))):
        n = na[0] + na[1]
        d8 = da[0, :, :HEADS] + da[1, :, :HEADS]
        g = n.reshape(_BLK, HEADS, OPH) / (d8[..., None] + 1e-16)
        g = g.reshape(_BLK, D) + gb[...]
        xk = jnp.dot(g, dw[...], preferred_element_type=jnp.float32) + db[...]
        xk = jnp.where(xk >= 0, xk, 0.01 * xk)
        acc = acc + DECAY[k] * xk
    mu = jnp.mean(acc, axis=-1, keepdims=True)
    var = jnp.mean((acc - mu) ** 2, axis=-1, keepdims=True)
    xl = (acc - mu) * lax.rsqrt(var + 1e-5) * lng[...] + lnb[...]
    hout[...] = xl + hres[...]


def _tcb(num0, den0, num1, den1, gb0, dw0, db0, gb1, dw1, db1, lng, lnb, h):
    nspec = pl.BlockSpec((NC, _BLK, D), lambda i: (0, i, 0))
    dspec = pl.BlockSpec((NC, _BLK, LN), lambda i: (0, i, 0))
    row = pl.BlockSpec((_BLK, D), lambda i: (i, 0))
    wfull = pl.BlockSpec((D, D), lambda i: (0, 0))
    vfull = pl.BlockSpec((1, D), lambda i: (0, 0))
    return pl.pallas_call(
        _tcb_body,
        grid=(_GRID,),
        in_specs=[nspec, dspec, nspec, dspec,
                  vfull, wfull, vfull, vfull, wfull, vfull,
                  vfull, vfull, row],
        out_specs=row,
        out_shape=jax.ShapeDtypeStruct((NPAD, D), jnp.float32),
    )(num0, den0, num1, den1, gb0, dw0, db0, gb1, dw1, db1, lng, lnb, h)


def _pad_edges(a, total, fill, width):
    return jnp.concatenate(
        [a, jnp.full((total - a.shape[0],), fill, jnp.int32)]
    ).reshape(-1, width)


def kernel(x, edge_index, edge_type, genre, genre_mask, W1, b1, gat_W,
           att_src, att_dst, gat_b, dec_W, dec_b, ln_g, ln_b):
    del edge_type, genre, genre_mask
    src = edge_index[0].astype(jnp.int32)
    dst = edge_index[1].astype(jnp.int32)
    loop = jnp.arange(N, dtype=jnp.int32)

    srcE = _pad_edges(src, K1PAD, DUMMY, ECHUNK)
    dstE = _pad_edges(dst, K1PAD, 0, ECHUNK)
    loc = _k1(srcE, dstE)
    nbr = _k2(loc)
    dst2 = _k3(nbr, dstE).reshape(-1)[:E]

    src_c = _pad_edges(jnp.concatenate([src, loop]), EPAD, DUMMY, ECONV)
    dst_c0 = _pad_edges(jnp.concatenate([dst, loop]), EPAD, DUMMY, ECONV)
    dst_c1 = _pad_edges(jnp.concatenate([dst2, loop]), EPAD, DUMMY, ECONV)

    xp = jnp.zeros((NPAD, D), jnp.float32).at[:N].set(x)
    h = _tc0(xp, W1, b1.reshape(1, D))

    r2 = lambda a: a.reshape(1, D)
    for l in range(LAYERS):
        h20, ss0, sd0, h21, ss1, sd1 = _tca(
            h, gat_W[l, 0], r2(att_src[l, 0]), r2(att_dst[l, 0]),
            gat_W[l, 1], r2(att_src[l, 1]), r2(att_dst[l, 1]))
        num0, den0, num1, den1 = _conv(src_c, dst_c0, ss0, sd0, h20,
                                       dst_c1, ss1, sd1, h21)
        h = _tcb(num0, den0, num1, den1,
                 r2(gat_b[l, 0]), dec_W[l, 0], r2(dec_b[l, 0]),
                 r2(gat_b[l, 1]), dec_W[l, 1], r2(dec_b[l, 1]),
                 r2(ln_g[l]), r2(ln_b[l]), h)
    return h[:N]
